# baseline (speedup 1.0000x reference)
import functools

import jax
import jax.numpy as jnp
from jax import lax
from jax.experimental import pallas as pl
from jax.experimental.pallas import tpu as pltpu
from jax.experimental.pallas import tpu_sc as plsc

_D = 64
_B = 4096 * 50 * 2
_NC = 2
_NS = 16
_NW = _NC * _NS
_BPW = _B // _NW
_CHUNK = 800
_NCH = _BPW // _CHUNK


def _gather_body(idx_hbm, table_hbm, out_hbm, idx_v, rows_v, gsem, wsem):
    wid = lax.axis_index("s") * _NC + lax.axis_index("c")
    base = wid * _BPW
    pltpu.sync_copy(idx_hbm.at[pl.ds(base, _BPW)], idx_v)

    def start_gather(c):
        s = c % 2
        return pltpu.async_copy(
            table_hbm.at[idx_v.at[pl.ds(c * _CHUNK, _CHUNK)]],
            rows_v.at[s],
            gsem.at[s],
        )

    g = [None] * _NCH
    w = [None] * _NCH
    g[0] = start_gather(0)
    for c in range(_NCH):
        s = c % 2
        if c >= 1:
            w[c - 1].wait()
        if c + 1 < _NCH:
            g[c + 1] = start_gather(c + 1)
        g[c].wait()
        w[c] = pltpu.async_copy(
            rows_v.at[s],
            out_hbm.at[pl.ds(base + c * _CHUNK, _CHUNK)],
            wsem.at[s],
        )
    w[_NCH - 1].wait()


@jax.jit
def _embed_lookup(idx_flat, table):
    mesh = plsc.VectorSubcoreMesh(core_axis_name="c", subcore_axis_name="s")
    run = pl.kernel(
        _gather_body,
        out_type=jax.ShapeDtypeStruct((_B, _D), jnp.float32),
        mesh=mesh,
        scratch_types=[
            pltpu.VMEM((_BPW,), jnp.int32),
            pltpu.VMEM((2, _CHUNK, _D), jnp.float32),
            pltpu.SemaphoreType.DMA((2,)),
            pltpu.SemaphoreType.DMA((2,)),
        ],
        compiler_params=pltpu.CompilerParams(use_tc_tiling_on_sc=False),
    )
    return run(idx_flat, table)


def kernel(idx, embedding_weight):
    idx_flat = idx.reshape(_B)
    out = _embed_lookup(idx_flat, embedding_weight)
    return out.reshape(idx.shape + (_D,))
